# baseline (speedup 1.0000x reference)
import functools

import jax
import jax.numpy as jnp
from jax import lax
from jax.experimental import pallas as pl
from jax.experimental.pallas import tpu as pltpu
from jax.experimental.pallas import tpu_sc as plsc


def _make_sc_broadcast(S, N, D, dtype):
    info = plsc.get_sparse_core_info()
    num_workers = info.num_cores * info.num_subcores
    rows_per_w = S // num_workers
    chunk = min(16, rows_per_w)
    n_chunks = rows_per_w // chunk
    mesh = plsc.VectorSubcoreMesh(core_axis_name="c", subcore_axis_name="s")

    @functools.partial(
        pl.kernel,
        mesh=mesh,
        out_type=jax.ShapeDtypeStruct((S, N, D), dtype),
        scratch_types=[
            pltpu.VMEM((chunk, D), dtype),
            pltpu.VMEM((chunk, D), dtype),
            pltpu.SemaphoreType.DMA,
            pltpu.SemaphoreType.DMA,
            pltpu.SemaphoreType.DMA,
            pltpu.SemaphoreType.DMA,
        ],
    )
    def sc_kernel(table_hbm, out_hbm, buf0, buf1, rsem0, rsem1, wsem0, wsem1):
        wid = lax.axis_index("s") * info.num_cores + lax.axis_index("c")
        base = wid * rows_per_w
        bufs, rsems, wsems = [buf0, buf1], [rsem0, rsem1], [wsem0, wsem1]

        def src(i):
            return table_hbm.at[pl.ds(base + i * chunk, chunk)]

        reads = {
            0: pltpu.async_copy(src(0), buf0, rsem0),
            1: pltpu.async_copy(src(1), buf1, rsem1),
        }
        tail_writes = []
        for i in range(n_chunks):
            b = i % 2
            reads[i].wait()
            writes = [
                pltpu.async_copy(
                    bufs[b], out_hbm.at[pl.ds(base + i * chunk, chunk), n], wsems[b]
                )
                for n in range(N)
            ]
            if i + 2 < n_chunks:
                for h in writes:
                    h.wait()
                reads[i + 2] = pltpu.async_copy(src(i + 2), bufs[b], rsems[b])
            else:
                tail_writes.extend(writes)
        for h in tail_writes:
            h.wait()

    return sc_kernel


def kernel(x, pos_embedding):
    S, N = x.shape
    _, D = pos_embedding.shape
    return _make_sc_broadcast(S, N, D, pos_embedding.dtype)(pos_embedding)


# speedup vs baseline: 1.1105x; 1.1105x over previous
import functools

import jax
import jax.numpy as jnp
from jax import lax
from jax.experimental import pallas as pl
from jax.experimental.pallas import tpu as pltpu
from jax.experimental.pallas import tpu_sc as plsc


def _chunk_sizes(rows, cap):
    sizes = [cap] * (rows // cap)
    if rows % cap:
        sizes.append(rows % cap)
    return sizes


def _make_sc_broadcast(S, N, D, dtype):
    info = plsc.get_sparse_core_info()
    num_workers = info.num_cores * info.num_subcores
    rows_per_w = S // num_workers
    cap = min(rows_per_w, (131071 // (2 * D)) // 8 * 8)
    sizes = _chunk_sizes(rows_per_w, cap)
    offs = [sum(sizes[:i]) for i in range(len(sizes))]
    n_chunks = len(sizes)
    mesh = plsc.VectorSubcoreMesh(core_axis_name="c", subcore_axis_name="s")

    @functools.partial(
        pl.kernel,
        mesh=mesh,
        out_type=jax.ShapeDtypeStruct((S, N, D), dtype),
        scratch_types=[
            pltpu.VMEM((cap, D), dtype),
            pltpu.VMEM((cap, D), dtype),
            pltpu.SemaphoreType.DMA,
            pltpu.SemaphoreType.DMA,
            pltpu.SemaphoreType.DMA,
            pltpu.SemaphoreType.DMA,
        ],
    )
    def sc_kernel(table_hbm, out_hbm, buf0, buf1, rsem0, rsem1, wsem0, wsem1):
        wid = lax.axis_index("s") * info.num_cores + lax.axis_index("c")
        base = wid * rows_per_w
        bufs, rsems, wsems = [buf0, buf1], [rsem0, rsem1], [wsem0, wsem1]

        def read(i):
            b = i % 2
            return pltpu.async_copy(
                table_hbm.at[pl.ds(base + offs[i], sizes[i])],
                bufs[b].at[pl.ds(0, sizes[i])],
                rsems[b],
            )

        reads = {j: read(j) for j in range(min(2, n_chunks))}
        tail_writes = []
        for i in range(n_chunks):
            b = i % 2
            reads[i].wait()
            writes = [
                pltpu.async_copy(
                    bufs[b].at[pl.ds(0, sizes[i])],
                    out_hbm.at[pl.ds(base + offs[i], sizes[i]), n],
                    wsems[b],
                )
                for n in range(N)
            ]
            if i + 2 < n_chunks:
                for h in writes:
                    h.wait()
                reads[i + 2] = read(i + 2)
            else:
                tail_writes.extend(writes)
        for h in tail_writes:
            h.wait()

    return sc_kernel


def kernel(x, pos_embedding):
    S, N = x.shape
    _, D = pos_embedding.shape
    return _make_sc_broadcast(S, N, D, pos_embedding.dtype)(pos_embedding)


# speedup vs baseline: 1.1169x; 1.0058x over previous
import functools

import jax
import jax.numpy as jnp
from jax import lax
from jax.experimental import pallas as pl
from jax.experimental.pallas import tpu as pltpu
from jax.experimental.pallas import tpu_sc as plsc


def _chunk_sizes(rows, cap):
    sizes = [cap] * (rows // cap)
    if rows % cap:
        sizes.append(rows % cap)
    return sizes


def _make_sc_broadcast(S, N, D, dtype):
    info = plsc.get_sparse_core_info()
    num_workers = info.num_cores * info.num_subcores
    rows_per_w = S // num_workers
    cap = min(rows_per_w, (131071 // D) // 8 * 8)
    sizes = _chunk_sizes(rows_per_w, cap)
    offs = [sum(sizes[:i]) for i in range(len(sizes))]
    n_chunks = len(sizes)
    mesh = plsc.VectorSubcoreMesh(core_axis_name="c", subcore_axis_name="s")

    @functools.partial(
        pl.kernel,
        mesh=mesh,
        out_type=jax.ShapeDtypeStruct((S, N, D), dtype),
        scratch_types=[
            pltpu.VMEM((cap, D), dtype),
            pltpu.SemaphoreType.DMA,
        ],
    )
    def sc_kernel(table_hbm, out_hbm, buf, wsem):
        wid = lax.axis_index("s") * info.num_cores + lax.axis_index("c")
        base = wid * rows_per_w

        for i in range(n_chunks):
            pltpu.sync_copy(
                table_hbm.at[pl.ds(base + offs[i], sizes[i])],
                buf.at[pl.ds(0, sizes[i])],
            )
            writes = [
                pltpu.async_copy(
                    buf.at[pl.ds(0, sizes[i])],
                    out_hbm.at[pl.ds(base + offs[i], sizes[i]), n],
                    wsem,
                )
                for n in range(N)
            ]
            for h in writes:
                h.wait()

    return sc_kernel


def kernel(x, pos_embedding):
    S, N = x.shape
    _, D = pos_embedding.shape
    return _make_sc_broadcast(S, N, D, pos_embedding.dtype)(pos_embedding)
